# baseline (speedup 1.0000x reference)
import functools

import jax
import jax.numpy as jnp
from jax import lax
from jax.experimental import pallas as pl
from jax.experimental.pallas import tpu as pltpu
from jax.experimental.pallas import tpu_sc as plsc

B, N, E, P = 4, 256, 8192, 512
NODE_F, EDGE_F, MSG = 256, 16, 128
H = 128

_NC, _NS = 2, 16
_NW = _NC * _NS
_PAIRS = B * P
_PPW = _PAIRS // _NW
_TPB = _NW // B
_ROWS = B * N * N // 8


def _sc_gather_body(pp_hbm, iff_hbm, s_hbm,
                    i0_v, i1_v, idxa_v, idxb_v, ga_v, gb_v, sum_v, sema, semb):
    c = lax.axis_index("c")
    s = lax.axis_index("s")
    wid = s * _NC + c
    base = wid * _PPW
    pltpu.sync_copy(pp_hbm.at[pl.ds(base, _PPW)], i0_v)
    pltpu.sync_copy(pp_hbm.at[pl.ds(_PAIRS + base, _PPW)], i1_v)
    bbase = (wid // _TPB) * N * 4096
    lane = lax.broadcasted_iota(jnp.int32, (16,), 0)
    for k in range(_PPW // 16):
        a = i0_v[pl.ds(k * 16, 16)]
        b = i1_v[pl.ds(k * 16, 16)]
        c0a = bbase + a * 4096 + (b >> 7) * 1024 + (b & 127)
        c0b = bbase + b * 4096 + (a >> 7) * 1024 + (a & 127)
        for f in range(EDGE_F):
            foff = (f // 8) * 2048 + (f % 8) * 128
            pos = lane * EDGE_F + (k * 256 + f)
            plsc.store_scatter(idxa_v, [pos], c0a + foff)
            plsc.store_scatter(idxb_v, [pos], c0b + foff)
    ca = pltpu.async_copy(iff_hbm.at[idxa_v], ga_v, sema)
    cb = pltpu.async_copy(iff_hbm.at[idxb_v], gb_v, semb)
    ca.wait()
    cb.wait()
    for q in range(_PPW * EDGE_F // 16):
        sum_v[q // 8, pl.ds((q % 8) * 16, 16)] = (
            ga_v[pl.ds(q * 16, 16)] + gb_v[pl.ds(q * 16, 16)])
    pltpu.sync_copy(sum_v, s_hbm.at[pl.ds(wid * 8, 8)])


@functools.lru_cache(maxsize=1)
def _sc_gather_kernel():
    return pl.kernel(
        _sc_gather_body,
        out_type=jax.ShapeDtypeStruct((_PAIRS * EDGE_F // 128, 128),
                                      jnp.float32),
        mesh=plsc.VectorSubcoreMesh(core_axis_name="c", subcore_axis_name="s"),
        scratch_types=[
            pltpu.VMEM((_PPW,), jnp.int32),
            pltpu.VMEM((_PPW,), jnp.int32),
            pltpu.VMEM((_PPW * EDGE_F,), jnp.int32),
            pltpu.VMEM((_PPW * EDGE_F,), jnp.int32),
            pltpu.VMEM((_PPW * EDGE_F,), jnp.float32),
            pltpu.VMEM((_PPW * EDGE_F,), jnp.float32),
            pltpu.VMEM((8, 128), jnp.float32),
            pltpu.SemaphoreType.DMA,
            pltpu.SemaphoreType.DMA,
        ],
        compiler_params=pltpu.CompilerParams(needs_layout_passes=False),
    )


def _tc_body(cnf_ref, ei_ref, pairs_ref, s_ref,
             wn_ref, bn_ref, we_ref, be_ref,
             wg1_ref, bg1_ref, wg2_ref, bg2_ref,
             wlr1_ref, blr1_ref, wlr2_ref, blr2_ref,
             wcr1_ref, bcr1_ref, wcr2_ref, bcr2_ref,
             wmr1_ref, bmr1_ref, wmr2_ref, bmr2_ref,
             lr_ref, cr_ref, mr_ref):
    f32 = jnp.float32
    x = cnf_ref[0]
    src = ei_ref[0, 0, :]
    dst = ei_ref[0, 1, :]
    cols = lax.broadcasted_iota(jnp.int32, (E, N), 1)
    s_oh = (src[:, None] == cols).astype(jnp.bfloat16)
    d_oh = (dst[:, None] == cols).astype(jnp.bfloat16)
    adj = lax.dot_general(d_oh, s_oh, (((0,), (0,)), ((), ())),
                          preferred_element_type=f32)
    inv_deg = 1.0 / (jnp.sum(adj, axis=1, keepdims=True) + 1.0)

    def gcn(h, w_ref, b_ref):
        y = jnp.dot(h, w_ref[...], preferred_element_type=f32)
        z = (jnp.dot(adj, y, preferred_element_type=f32) + y) * inv_deg
        return jax.nn.relu(z + b_ref[...])

    h1 = gcn(x, wg1_ref, bg1_ref)
    node_emb = gcn(h1, wg2_ref, bg2_ref)
    obj_ft = jnp.dot(x, wn_ref[...], preferred_element_type=f32) + bn_ref[...]
    half = 0.5 * (node_emb + obj_ft)

    i0 = pairs_ref[0, :, 0]
    i1 = pairs_ref[0, :, 1]
    pcols = lax.broadcasted_iota(jnp.int32, (P, N), 1)
    p0 = (i0[:, None] == pcols).astype(f32)
    p1 = (i1[:, None] == pcols).astype(f32)
    t0 = jnp.dot(p0, half, preferred_element_type=f32)
    t1 = jnp.dot(p1, half, preferred_element_type=f32)

    spk = s_ref[...]
    ge = 0.5 * jnp.concatenate(
        [spk[:, k * EDGE_F:(k + 1) * EDGE_F] for k in range(8)], axis=0)
    te = jnp.dot(ge, we_ref[...], preferred_element_type=f32) + be_ref[...]

    def head(w1_ref, b1_ref, w2_ref, b2_ref, o_ref):
        w1 = w1_ref[...]
        hid = (jnp.dot(t0, w1[0:MSG, :], preferred_element_type=f32)
               + jnp.dot(t1, w1[MSG:2 * MSG, :], preferred_element_type=f32)
               + jnp.dot(te, w1[2 * MSG:3 * MSG, :], preferred_element_type=f32)
               + b1_ref[...])
        o_ref[0] = jnp.dot(jax.nn.relu(hid), w2_ref[...],
                           preferred_element_type=f32) + b2_ref[...]

    head(wlr1_ref, blr1_ref, wlr2_ref, blr2_ref, lr_ref)
    head(wcr1_ref, bcr1_ref, wcr2_ref, bcr2_ref, cr_ref)
    head(wmr1_ref, bmr1_ref, wmr2_ref, bmr2_ref, mr_ref)


@functools.lru_cache(maxsize=1)
def _tc_forward():
    full = lambda shp: pl.BlockSpec(shp, lambda b: (0,) * len(shp))
    grid_spec = pl.GridSpec(
        grid=(B,),
        in_specs=[
            pl.BlockSpec((1, N, NODE_F), lambda b: (b, 0, 0)),
            pl.BlockSpec((1, 2, E), lambda b: (b, 0, 0)),
            pl.BlockSpec((1, P, 2), lambda b: (b, 0, 0)),
            pl.BlockSpec((P * EDGE_F // 128, 128), lambda b: (b, 0)),
            full((NODE_F, MSG)), full((MSG,)),
            full((EDGE_F, MSG)), full((MSG,)),
            full((NODE_F, MSG)), full((MSG,)),
            full((MSG, MSG)), full((MSG,)),
            full((3 * MSG, H)), full((H,)), full((H, 7)), full((7,)),
            full((3 * MSG, H)), full((H,)), full((H, 5)), full((5,)),
            full((3 * MSG, H)), full((H,)), full((H, 3)), full((3,)),
        ],
        out_specs=[
            pl.BlockSpec((1, P, 7), lambda b: (b, 0, 0)),
            pl.BlockSpec((1, P, 5), lambda b: (b, 0, 0)),
            pl.BlockSpec((1, P, 3), lambda b: (b, 0, 0)),
        ],
    )
    return pl.pallas_call(
        _tc_body,
        grid_spec=grid_spec,
        out_shape=[
            jax.ShapeDtypeStruct((B, P, 7), jnp.float32),
            jax.ShapeDtypeStruct((B, P, 5), jnp.float32),
            jax.ShapeDtypeStruct((B, P, 3), jnp.float32),
        ],
    )


def kernel(concatenated_node_features, interaction_feature, edge_index,
           object_pairs, W_node, b_node, W_edge, b_edge, W_g1, b_g1,
           W_g2, b_g2, W_lr1, b_lr1, W_lr2, b_lr2, W_cr1, b_cr1,
           W_cr2, b_cr2, W_mr1, b_mr1, W_mr2, b_mr2):
    iff1d = (interaction_feature
             .transpose(0, 1, 3, 2)
             .reshape(B, N, 2, 8, 2, 128)
             .transpose(0, 1, 2, 4, 3, 5)
             .reshape(B * N * N * EDGE_F))
    pp = (object_pairs.reshape(B, 8, 64, 2).transpose(0, 2, 1, 3)
          .reshape(B * P, 2))
    pairs_pack = jnp.concatenate([pp[:, 0], pp[:, 1]])
    s_pack = _sc_gather_kernel()(pairs_pack, iff1d)
    lr, cr, mr = _tc_forward()(
        concatenated_node_features, edge_index, object_pairs, s_pack,
        W_node, b_node, W_edge, b_edge, W_g1, b_g1, W_g2, b_g2,
        W_lr1, b_lr1, W_lr2, b_lr2, W_cr1, b_cr1, W_cr2, b_cr2,
        W_mr1, b_mr1, W_mr2, b_mr2)
    return (lr, cr, mr)


# speedup vs baseline: 1.0540x; 1.0540x over previous
import functools

import jax
import jax.numpy as jnp
from jax import lax
from jax.experimental import pallas as pl
from jax.experimental.pallas import tpu as pltpu
from jax.experimental.pallas import tpu_sc as plsc

B, N, E, P = 4, 256, 8192, 512
NODE_F, EDGE_F, MSG = 256, 16, 128
H = 128

_NC, _NS = 2, 16
_NW = _NC * _NS
_PAIRS = B * P
_PPW = _PAIRS // _NW
_TPB = _NW // B
_ROWS = B * N * N // 8


def _sc_gather_body(pp_hbm, iff_hbm, s_hbm,
                    i0_v, i1_v, idxa_v, idxb_v, ga_v, gb_v, sum_v, sema, semb):
    c = lax.axis_index("c")
    s = lax.axis_index("s")
    wid = s * _NC + c
    base = wid * _PPW
    pltpu.sync_copy(pp_hbm.at[pl.ds(base, _PPW)], i0_v)
    pltpu.sync_copy(pp_hbm.at[pl.ds(_PAIRS + base, _PPW)], i1_v)
    bbase = (wid // _TPB) * N * 4096
    lane = lax.broadcasted_iota(jnp.int32, (16,), 0)
    for k in range(_PPW // 16):
        a = i0_v[pl.ds(k * 16, 16)]
        b = i1_v[pl.ds(k * 16, 16)]
        c0a = bbase + a * 4096 + (b >> 7) * 1024 + (b & 127)
        c0b = bbase + b * 4096 + (a >> 7) * 1024 + (a & 127)
        for f in range(EDGE_F):
            foff = (f // 8) * 2048 + (f % 8) * 128
            pos = lane * EDGE_F + (k * 256 + f)
            plsc.store_scatter(idxa_v, [pos], c0a + foff)
            plsc.store_scatter(idxb_v, [pos], c0b + foff)
    ca = pltpu.async_copy(iff_hbm.at[idxa_v], ga_v, sema)
    cb = pltpu.async_copy(iff_hbm.at[idxb_v], gb_v, semb)
    ca.wait()
    cb.wait()
    for q in range(_PPW * EDGE_F // 16):
        sum_v[q // 8, pl.ds((q % 8) * 16, 16)] = (
            ga_v[pl.ds(q * 16, 16)] + gb_v[pl.ds(q * 16, 16)])
    pltpu.sync_copy(sum_v, s_hbm.at[pl.ds(wid * 8, 8)])


@functools.lru_cache(maxsize=1)
def _sc_gather_kernel():
    return pl.kernel(
        _sc_gather_body,
        out_type=jax.ShapeDtypeStruct((_PAIRS * EDGE_F // 128, 128),
                                      jnp.float32),
        mesh=plsc.VectorSubcoreMesh(core_axis_name="c", subcore_axis_name="s"),
        scratch_types=[
            pltpu.VMEM((_PPW,), jnp.int32),
            pltpu.VMEM((_PPW,), jnp.int32),
            pltpu.VMEM((_PPW * EDGE_F,), jnp.int32),
            pltpu.VMEM((_PPW * EDGE_F,), jnp.int32),
            pltpu.VMEM((_PPW * EDGE_F,), jnp.float32),
            pltpu.VMEM((_PPW * EDGE_F,), jnp.float32),
            pltpu.VMEM((8, 128), jnp.float32),
            pltpu.SemaphoreType.DMA,
            pltpu.SemaphoreType.DMA,
        ],
        compiler_params=pltpu.CompilerParams(needs_layout_passes=False),
    )


def _tc1_body(cnf_ref, ei_ref, wn_ref, bn_ref,
              wg1_ref, bg1_ref, wg2_ref, bg2_ref, half_ref):
    f32 = jnp.float32
    x = cnf_ref[0]
    src = ei_ref[0, 0, :]
    dst = ei_ref[0, 1, :]
    cols = lax.broadcasted_iota(jnp.int32, (E, N), 1)
    s_oh = (src[:, None] == cols).astype(jnp.bfloat16)
    d_oh = (dst[:, None] == cols).astype(jnp.bfloat16)
    adj = lax.dot_general(d_oh, s_oh, (((0,), (0,)), ((), ())),
                          preferred_element_type=f32)
    inv_deg = 1.0 / (jnp.sum(adj, axis=1, keepdims=True) + 1.0)

    def gcn(h, w_ref, b_ref):
        y = jnp.dot(h, w_ref[...], preferred_element_type=f32)
        z = (jnp.dot(adj, y, preferred_element_type=f32) + y) * inv_deg
        return jax.nn.relu(z + b_ref[...])

    h1 = gcn(x, wg1_ref, bg1_ref)
    node_emb = gcn(h1, wg2_ref, bg2_ref)
    obj_ft = jnp.dot(x, wn_ref[...], preferred_element_type=f32) + bn_ref[...]
    half_ref[0] = 0.5 * (node_emb + obj_ft)


def _tc2_body(pairs_ref, s_ref, half_ref, we_ref, be_ref,
              wlr1_ref, blr1_ref, wlr2_ref, blr2_ref,
              wcr1_ref, bcr1_ref, wcr2_ref, bcr2_ref,
              wmr1_ref, bmr1_ref, wmr2_ref, bmr2_ref,
              lr_ref, cr_ref, mr_ref):
    f32 = jnp.float32
    half = half_ref[0]
    i0 = pairs_ref[0, :, 0]
    i1 = pairs_ref[0, :, 1]
    pcols = lax.broadcasted_iota(jnp.int32, (P, N), 1)
    p0 = (i0[:, None] == pcols).astype(f32)
    p1 = (i1[:, None] == pcols).astype(f32)
    t0 = jnp.dot(p0, half, preferred_element_type=f32)
    t1 = jnp.dot(p1, half, preferred_element_type=f32)

    spk = s_ref[...]
    ge = 0.5 * jnp.concatenate(
        [spk[:, k * EDGE_F:(k + 1) * EDGE_F] for k in range(8)], axis=0)
    te = jnp.dot(ge, we_ref[...], preferred_element_type=f32) + be_ref[...]

    def head(w1_ref, b1_ref, w2_ref, b2_ref, o_ref):
        w1 = w1_ref[...]
        hid = (jnp.dot(t0, w1[0:MSG, :], preferred_element_type=f32)
               + jnp.dot(t1, w1[MSG:2 * MSG, :], preferred_element_type=f32)
               + jnp.dot(te, w1[2 * MSG:3 * MSG, :], preferred_element_type=f32)
               + b1_ref[...])
        o_ref[0] = jnp.dot(jax.nn.relu(hid), w2_ref[...],
                           preferred_element_type=f32) + b2_ref[...]

    head(wlr1_ref, blr1_ref, wlr2_ref, blr2_ref, lr_ref)
    head(wcr1_ref, bcr1_ref, wcr2_ref, bcr2_ref, cr_ref)
    head(wmr1_ref, bmr1_ref, wmr2_ref, bmr2_ref, mr_ref)


@functools.lru_cache(maxsize=1)
def _tc1():
    full = lambda shp: pl.BlockSpec(shp, lambda b: (0,) * len(shp))
    grid_spec = pl.GridSpec(
        grid=(B,),
        in_specs=[
            pl.BlockSpec((1, N, NODE_F), lambda b: (b, 0, 0)),
            pl.BlockSpec((1, 2, E), lambda b: (b, 0, 0)),
            full((NODE_F, MSG)), full((MSG,)),
            full((NODE_F, MSG)), full((MSG,)),
            full((MSG, MSG)), full((MSG,)),
        ],
        out_specs=pl.BlockSpec((1, N, MSG), lambda b: (b, 0, 0)),
    )
    return pl.pallas_call(
        _tc1_body,
        grid_spec=grid_spec,
        out_shape=jax.ShapeDtypeStruct((B, N, MSG), jnp.float32),
    )


@functools.lru_cache(maxsize=1)
def _tc2():
    full = lambda shp: pl.BlockSpec(shp, lambda b: (0,) * len(shp))
    grid_spec = pl.GridSpec(
        grid=(B,),
        in_specs=[
            pl.BlockSpec((1, P, 2), lambda b: (b, 0, 0)),
            pl.BlockSpec((P * EDGE_F // 128, 128), lambda b: (b, 0)),
            pl.BlockSpec((1, N, MSG), lambda b: (b, 0, 0)),
            full((EDGE_F, MSG)), full((MSG,)),
            full((3 * MSG, H)), full((H,)), full((H, 7)), full((7,)),
            full((3 * MSG, H)), full((H,)), full((H, 5)), full((5,)),
            full((3 * MSG, H)), full((H,)), full((H, 3)), full((3,)),
        ],
        out_specs=[
            pl.BlockSpec((1, P, 7), lambda b: (b, 0, 0)),
            pl.BlockSpec((1, P, 5), lambda b: (b, 0, 0)),
            pl.BlockSpec((1, P, 3), lambda b: (b, 0, 0)),
        ],
    )
    return pl.pallas_call(
        _tc2_body,
        grid_spec=grid_spec,
        out_shape=[
            jax.ShapeDtypeStruct((B, P, 7), jnp.float32),
            jax.ShapeDtypeStruct((B, P, 5), jnp.float32),
            jax.ShapeDtypeStruct((B, P, 3), jnp.float32),
        ],
    )


def kernel(concatenated_node_features, interaction_feature, edge_index,
           object_pairs, W_node, b_node, W_edge, b_edge, W_g1, b_g1,
           W_g2, b_g2, W_lr1, b_lr1, W_lr2, b_lr2, W_cr1, b_cr1,
           W_cr2, b_cr2, W_mr1, b_mr1, W_mr2, b_mr2):
    iff1d = (interaction_feature
             .transpose(0, 1, 3, 2)
             .reshape(B, N, 2, 8, 2, 128)
             .transpose(0, 1, 2, 4, 3, 5)
             .reshape(B * N * N * EDGE_F))
    pp = (object_pairs.reshape(B, 8, 64, 2).transpose(0, 2, 1, 3)
          .reshape(B * P, 2))
    pairs_pack = jnp.concatenate([pp[:, 0], pp[:, 1]])
    s_pack = _sc_gather_kernel()(pairs_pack, iff1d)
    half = _tc1()(concatenated_node_features, edge_index,
                  W_node, b_node, W_g1, b_g1, W_g2, b_g2)
    lr, cr, mr = _tc2()(
        object_pairs, s_pack, half, W_edge, b_edge,
        W_lr1, b_lr1, W_lr2, b_lr2, W_cr1, b_cr1, W_cr2, b_cr2,
        W_mr1, b_mr1, W_mr2, b_mr2)
    return (lr, cr, mr)
